# baseline (speedup 1.0000x reference)
import functools

import jax
import jax.numpy as jnp
from jax import lax
from jax.experimental import pallas as pl
from jax.experimental.pallas import tpu as pltpu
from jax.experimental.pallas import tpu_sc as plsc

N = 50000
K = 16
WIDTH = 64
CUTOFF = 3.0

BNA = 2000
BNB = 1000
CH1 = 5000
CH2 = 1000

NBS = N // BNB


def _sc_gather_wide(table, idxg, cw):
    M, D = table.shape
    R = idxg.shape[1]
    n_chunks = R // cw
    info = plsc.get_sparse_core_info()
    nw = info.num_cores * info.num_subcores
    n_iter = (n_chunks + nw - 1) // nw
    mesh = plsc.VectorSubcoreMesh(core_axis_name="c", subcore_axis_name="s")

    @functools.partial(
        pl.kernel,
        mesh=mesh,
        out_type=jax.ShapeDtypeStruct((R, 8 * D), table.dtype),
        scratch_types=[
            pltpu.VMEM((8, cw), jnp.int32),
            pltpu.VMEM((8, cw, D), table.dtype),
            pltpu.SemaphoreType.DMA,
        ],
        compiler_params=pltpu.CompilerParams(use_tc_tiling_on_sc=False),
    )
    def k(table_hbm, idxg_hbm, out_hbm, idx_v, rows_v, sem):
        wid = lax.axis_index("s") * info.num_cores + lax.axis_index("c")

        def body(i, carry):
            g = i * nw + wid

            @pl.when(g < n_chunks)
            def _():
                base = g * cw
                pltpu.sync_copy(idxg_hbm.at[:, pl.ds(base, cw)], idx_v)
                descs = [
                    pltpu.async_copy(table_hbm.at[idx_v.at[j]],
                                     rows_v.at[j], sem)
                    for j in range(8)
                ]
                for dsc in descs:
                    dsc.wait()
                for j in range(8):
                    pltpu.sync_copy(
                        rows_v.at[j],
                        out_hbm.at[pl.ds(base, cw), pl.ds(j * D, D)])

            return carry

        lax.fori_loop(0, n_iter, body, 0)

    return k(table, idxg)


def _silu(x):
    return x * (1.0 / (1.0 + jnp.exp(-x)))


def _t1a_body(rcp_ref, rnbp_ref, w2_ref, ob8_ref, scl_ref, cbc_ref, eyeb_ref,
              mask_ref, EWd_ref, envbd_ref, D1d_ref, d1bd_ref, D2d_ref,
              d2bd_ref, beta_ref, gp_ref):
    diffp = rcp_ref[...] - rnbp_ref[...]
    sq = diffp * diffp

    inv_s = 1.0 / jnp.log1p(jnp.exp(scl_ref[...]))
    inv_sP = jnp.concatenate([inv_s] * 8, axis=1)
    CCW = jnp.concatenate([cbc_ref[...], eyeb_ref[...] * inv_sP], axis=1)
    CC = sq @ CCW
    dist2bc = CC[:, :64]
    distbc = jnp.sqrt(dist2bc)
    env = jnp.exp(-CC[:, 64:])
    envp = env @ EWd_ref[...] + envbd_ref[...]

    d = _silu(diffp @ D1d_ref[...] + d1bd_ref[...])
    d = d @ D2d_ref[...] + d2bd_ref[...]

    q = distbc * (1.0 / CUTOFF)
    u = dist2bc * (1.0 / (CUTOFF * CUTOFF))
    u2 = u * u
    cut = jnp.where(u < 1.0, 1.0 + u2 * (24.0 * q - (15.0 + 10.0 * u)), 0.0)
    wv = w2_ref[...] @ ob8_ref[...]
    beta_ref[...] = envp * d * cut * wv
    gp_ref[...] = diffp + distbc * mask_ref[...]


def _t1b_body(gp0_ref, gp1_ref, b0_ref, b1_ref, W0d_ref, b0d_ref, W1_ref,
              b1d_ref, EGd_ref, mW0_ref, mb0_ref, mW1_ref, mb1_ref,
              mW2_ref, mb2_ref, hpre_ref, h_ref):
    def slab(gp_ref, beta_ref):
        x1 = _silu(gp_ref[...] @ W0d_ref[...] + b0d_ref[...])
        bg = beta_ref[...] @ EGd_ref[...]
        acc = None
        for j in range(4):
            x2 = _silu(x1[:, 128 * j:128 * (j + 1)] @ W1_ref[...]
                       + b1d_ref[...])
            p = x2 * bg[:, 128 * j:128 * (j + 1)]
            acc = p if acc is None else acc + p
        return acc

    s = slab(gp0_ref, b0_ref) + slab(gp1_ref, b1_ref)
    h0 = s[:, :WIDTH] + s[:, WIDTH:]

    y = h0 @ mW0_ref[...] + mb0_ref[...]
    h = _silu(y[:, :WIDTH])
    hpre_ref[...] = y[:, WIDTH:]
    h = _silu(h @ mW1_ref[...] + mb1_ref[...])
    h_ref[...] = (h @ mW2_ref[...] + mb2_ref[...]).astype(jnp.bfloat16)


def _t2_body(b0_ref, b1_ref, hnb0_ref, hnb1_ref, hpre_ref, r_ref, MGd_ref,
             Rt_ref, R2_ref, orbW_ref, orbb_ref, phi_ref):
    def slab(beta_ref, hnb_ref):
        bg = beta_ref[...] @ MGd_ref[...]
        hv = hnb_ref[...].astype(jnp.float32)
        acc = None
        for j in range(4):
            p = bg[:, 128 * j:128 * (j + 1)] * hv[:, 128 * j:128 * (j + 1)]
            acc = p if acc is None else acc + p
        return acc

    s = slab(b0_ref, hnb0_ref) + slab(b1_ref, hnb1_ref)
    msum = s[:, :WIDTH] + s[:, WIDTH:]
    h_out = _silu(hpre_ref[...] + msum)

    rc = r_ref[...]
    rr = jnp.sum(rc * rc, axis=1, keepdims=True)
    d2 = rr - 2.0 * (rc @ Rt_ref[...]) + R2_ref[...]
    deo = jnp.sqrt(jnp.maximum(d2, 0.0))
    phi_ref[...] = (h_out @ orbW_ref[...] + orbb_ref[...]) * jnp.exp(-0.2 * deo)


def _full(shape):
    nd = len(shape)
    return pl.BlockSpec(shape, lambda *_: (0,) * nd)


def _blockdiag(W, p):
    a, b = W.shape
    out = jnp.zeros((p * a, p * b), jnp.float32)
    for j in range(p):
        out = out.at[j * a:(j + 1) * a, j * b:(j + 1) * b].set(W)
    return out


def kernel(r, ind_neighbour, weight_neighbour, R_orb, beta_scales, beta_env_W,
           beta_env_b, beta_d1_W, beta_d1_b, beta_d2_W, beta_d2_b, emb_W0,
           emb_b0, emb_W1, emb_b1, emb_gamma_W, mlp_W0, mlp_b0, mlp_W1, mlp_b1,
           mlp_W2, mlp_b2, mp_proj_W, mp_proj_b, mp_gamma_W, orb_W, orb_b):
    E = N * K
    RW = 2 * N
    f32 = jnp.float32
    ind = ind_neighbour.astype(jnp.int32)
    idx = jnp.concatenate([ind[:, :8], ind[:, 8:]], axis=0)
    idxg = idx.T
    r8 = jnp.concatenate([r, jnp.zeros((N, 5), f32)], axis=1)
    r64 = jnp.tile(r8, (1, 8))
    w2 = jnp.concatenate(
        [weight_neighbour[:, :8], weight_neighbour[:, 8:]], axis=0)
    Rt8 = jnp.concatenate([R_orb.T, jnp.zeros((5, 32), f32)], axis=0)
    R2 = jnp.sum(R_orb * R_orb, axis=1)[None, :]

    tileb = lambda v, p: jnp.concatenate([v.reshape(1, -1)] * p, axis=1)
    row = lambda v: v.reshape(1, -1)
    cbc = _blockdiag(jnp.ones((8, 8), f32), 8)
    ob8 = _blockdiag(jnp.ones((1, 8), f32), 8)
    eyeb = _blockdiag(jnp.ones((8, 16), f32), 8)
    mask = jnp.zeros((1, 64), f32)
    for j in range(8):
        mask = mask.at[0, 8 * j + 3].set(1.0)
    d1W8 = jnp.concatenate([beta_d1_W, jnp.zeros((5, 16), f32)], axis=0)
    D1d = _blockdiag(d1W8, 8)
    D2d = _blockdiag(beta_d2_W, 8)
    EWd = _blockdiag(beta_env_W, 8)
    eW0_8 = jnp.concatenate([emb_W0, jnp.zeros((4, WIDTH), f32)], axis=0)
    W0d = _blockdiag(eW0_8, 8)
    EGd = _blockdiag(emb_gamma_W, 8)
    MGd = _blockdiag(mp_gamma_W, 8)

    rnbw = _sc_gather_wide(r8, idxg, 1000)

    nba = N // BNA
    beta8, gp8 = pl.pallas_call(
        _t1a_body,
        grid=(RW // BNA,),
        in_specs=[
            pl.BlockSpec((BNA, 64), lambda i: (i % nba, 0)),
            pl.BlockSpec((BNA, 64), lambda i: (i, 0)),
            pl.BlockSpec((BNA, 8), lambda i: (i, 0)),
            _full((8, 64)),
            _full((1, 16)), _full((64, 64)), _full((64, 128)), _full((1, 64)),
            _full((128, 64)), _full((1, 64)),
            _full((64, 128)), _full((1, 128)), _full((128, 64)), _full((1, 64)),
        ],
        out_specs=[
            pl.BlockSpec((BNA, 64), lambda i: (i, 0)),
            pl.BlockSpec((BNA, 64), lambda i: (i, 0)),
        ],
        out_shape=[
            jax.ShapeDtypeStruct((RW, 64), f32),
            jax.ShapeDtypeStruct((RW, 64), f32),
        ],
        compiler_params=pltpu.CompilerParams(
            dimension_semantics=("arbitrary",)),
    )(r64, rnbw, w2, ob8,
      row(beta_scales), cbc, eyeb, mask,
      EWd, tileb(beta_env_b, 8), D1d, tileb(beta_d1_b, 8),
      D2d, tileb(beta_d2_b, 8))

    slab0 = pl.BlockSpec((BNB, 64), lambda i: (i, 0))
    slab1 = pl.BlockSpec((BNB, 64), lambda i: (i + NBS, 0))
    hpre, h = pl.pallas_call(
        _t1b_body,
        grid=(N // BNB,),
        in_specs=[
            slab0, slab1, slab0, slab1,
            _full((64, 512)), _full((1, 512)), _full((128, 128)),
            _full((1, 128)), _full((64, 512)),
            _full((WIDTH, 128)), _full((1, 128)),
            _full((WIDTH, WIDTH)), _full((1, WIDTH)),
            _full((WIDTH, WIDTH)), _full((1, WIDTH)),
        ],
        out_specs=[
            pl.BlockSpec((BNB, WIDTH), lambda i: (i, 0)),
            pl.BlockSpec((BNB, WIDTH), lambda i: (i, 0)),
        ],
        out_shape=[
            jax.ShapeDtypeStruct((N, WIDTH), f32),
            jax.ShapeDtypeStruct((N, WIDTH), jnp.bfloat16),
        ],
        compiler_params=pltpu.CompilerParams(
            dimension_semantics=("arbitrary",)),
    )(gp8, gp8, beta8, beta8,
      W0d, tileb(emb_b0, 8), _blockdiag(emb_W1, 2), tileb(emb_b1, 2), EGd,
      jnp.concatenate([mlp_W0, mp_proj_W], axis=1),
      jnp.concatenate([row(mlp_b0), row(mp_proj_b)], axis=1),
      mlp_W1, row(mlp_b1), mlp_W2, row(mlp_b2))

    hnbw = _sc_gather_wide(h, idxg, 200)

    slab0w = pl.BlockSpec((BNB, 512), lambda i: (i, 0))
    slab1w = pl.BlockSpec((BNB, 512), lambda i: (i + NBS, 0))
    phi = pl.pallas_call(
        _t2_body,
        grid=(N // BNB,),
        in_specs=[
            slab0, slab1, slab0w, slab1w,
            pl.BlockSpec((BNB, WIDTH), lambda i: (i, 0)),
            pl.BlockSpec((BNB, 8), lambda i: (i, 0)),
            _full((64, 512)), _full((8, 32)), _full((1, 32)),
            _full((WIDTH, 32)), _full((1, 32)),
        ],
        out_specs=pl.BlockSpec((BNB, 32), lambda i: (i, 0)),
        out_shape=jax.ShapeDtypeStruct((N, 32), f32),
        compiler_params=pltpu.CompilerParams(
            dimension_semantics=("arbitrary",)),
    )(beta8, beta8, hnbw, hnbw, hpre, r8,
      MGd, Rt8, R2, orb_W, row(orb_b))

    return phi


# speedup vs baseline: 1.2107x; 1.2107x over previous
import functools

import jax
import jax.numpy as jnp
from jax import lax
from jax.experimental import pallas as pl
from jax.experimental.pallas import tpu as pltpu
from jax.experimental.pallas import tpu_sc as plsc

N = 50000
K = 16
WIDTH = 64
CUTOFF = 3.0

BNA = 2000
BNB = 1000
CH1 = 5000
CH2 = 1000

NBS = N // BNB


def _sc_gather_wide(table, idxg, cw):
    M, D = table.shape
    R = idxg.shape[1]
    n_chunks = R // cw
    info = plsc.get_sparse_core_info()
    nw = info.num_cores * info.num_subcores
    n_iter = (n_chunks + nw - 1) // nw
    mesh = plsc.VectorSubcoreMesh(core_axis_name="c", subcore_axis_name="s")

    @functools.partial(
        pl.kernel,
        mesh=mesh,
        out_type=jax.ShapeDtypeStruct((R, 8 * D), jnp.float32),
        scratch_types=[
            pltpu.VMEM((8, cw), jnp.int32),
            pltpu.VMEM((8, cw, D), jnp.float32),
            pltpu.SemaphoreType.DMA,
        ],
        compiler_params=pltpu.CompilerParams(use_tc_tiling_on_sc=False),
    )
    def k(table_hbm, idxg_hbm, out_hbm, idx_v, rows_v, sem):
        wid = lax.axis_index("s") * info.num_cores + lax.axis_index("c")

        def body(i, carry):
            g = i * nw + wid

            @pl.when(g < n_chunks)
            def _():
                base = g * cw
                pltpu.sync_copy(idxg_hbm.at[:, pl.ds(base, cw)], idx_v)
                descs = [
                    pltpu.async_copy(table_hbm.at[idx_v.at[j]],
                                     rows_v.at[j], sem)
                    for j in range(8)
                ]
                for dsc in descs:
                    dsc.wait()
                for j in range(8):
                    pltpu.sync_copy(
                        rows_v.at[j],
                        out_hbm.at[pl.ds(base, cw), pl.ds(j * D, D)])

            return carry

        lax.fori_loop(0, n_iter, body, 0)

    return k(table, idxg)


def _silu(x):
    return x * (1.0 / (1.0 + jnp.exp(-x)))


def _t1_body(r64_ref, rnb0_ref, rnb1_ref, w20_ref, w21_ref, ob8_ref,
             scl_ref, cbc_ref, eyeb_ref, mask_ref, EWd_ref, envbd_ref,
             D1d_ref, d1bd_ref, D2d_ref, d2bd_ref,
             W0d_ref, b0d_ref, W1_ref, b1d_ref, EGd_ref,
             mW0_ref, mb0_ref, mW1_ref, mb1_ref, mW2_ref, mb2_ref,
             beta0_ref, beta1_ref, hpre_ref, h_ref):
    inv_s = 1.0 / jnp.log1p(jnp.exp(scl_ref[...]))
    inv_sP = jnp.concatenate([inv_s] * 8, axis=1)
    CCW = jnp.concatenate([cbc_ref[...], eyeb_ref[...] * inv_sP], axis=1)
    rcp = r64_ref[...]

    def slab(rnb_ref, w2_ref, beta_ref):
        diffp = rcp - rnb_ref[...]
        sq = diffp * diffp
        CC = sq @ CCW
        dist2bc = CC[:, :64]
        distbc = jnp.sqrt(dist2bc)
        env = jnp.exp(-CC[:, 64:])
        envp = env @ EWd_ref[...] + envbd_ref[...]

        d = _silu(diffp @ D1d_ref[...] + d1bd_ref[...])
        d = d @ D2d_ref[...] + d2bd_ref[...]

        q = distbc * (1.0 / CUTOFF)
        u = dist2bc * (1.0 / (CUTOFF * CUTOFF))
        u2 = u * u
        cut = jnp.where(u < 1.0,
                        1.0 + u2 * (24.0 * q - (15.0 + 10.0 * u)), 0.0)
        wv = w2_ref[...] @ ob8_ref[...]
        beta = envp * d * cut * wv
        beta_ref[...] = beta
        gp = diffp + distbc * mask_ref[...]

        x1 = _silu(gp @ W0d_ref[...] + b0d_ref[...])
        bg = beta @ EGd_ref[...]
        acc = None
        for j in range(4):
            x2 = _silu(x1[:, 128 * j:128 * (j + 1)] @ W1_ref[...]
                       + b1d_ref[...])
            p = x2 * bg[:, 128 * j:128 * (j + 1)]
            acc = p if acc is None else acc + p
        return acc

    s = slab(rnb0_ref, w20_ref, beta0_ref) + slab(rnb1_ref, w21_ref, beta1_ref)
    h0 = s[:, :WIDTH] + s[:, WIDTH:]

    y = h0 @ mW0_ref[...] + mb0_ref[...]
    h = _silu(y[:, :WIDTH])
    hpre_ref[...] = y[:, WIDTH:]
    h = _silu(h @ mW1_ref[...] + mb1_ref[...])
    h_ref[...] = h @ mW2_ref[...] + mb2_ref[...]


def _t2_body(b0_ref, b1_ref, hnb0_ref, hnb1_ref, hpre_ref, r_ref, MGd_ref,
             Rt_ref, R2_ref, orbW_ref, orbb_ref, phi_ref):
    def slab(beta_ref, hnb_ref):
        bg = beta_ref[...] @ MGd_ref[...]
        hv = hnb_ref[...]
        acc = None
        for j in range(4):
            p = bg[:, 128 * j:128 * (j + 1)] * hv[:, 128 * j:128 * (j + 1)]
            acc = p if acc is None else acc + p
        return acc

    s = slab(b0_ref, hnb0_ref) + slab(b1_ref, hnb1_ref)
    msum = s[:, :WIDTH] + s[:, WIDTH:]
    h_out = _silu(hpre_ref[...] + msum)

    rc = r_ref[...]
    rr = jnp.sum(rc * rc, axis=1, keepdims=True)
    d2 = rr - 2.0 * (rc @ Rt_ref[...]) + R2_ref[...]
    deo = jnp.sqrt(jnp.maximum(d2, 0.0))
    phi_ref[...] = (h_out @ orbW_ref[...] + orbb_ref[...]) * jnp.exp(-0.2 * deo)


def _full(shape):
    nd = len(shape)
    return pl.BlockSpec(shape, lambda *_: (0,) * nd)


def _blockdiag(W, p):
    a, b = W.shape
    out = jnp.zeros((p * a, p * b), jnp.float32)
    for j in range(p):
        out = out.at[j * a:(j + 1) * a, j * b:(j + 1) * b].set(W)
    return out


def kernel(r, ind_neighbour, weight_neighbour, R_orb, beta_scales, beta_env_W,
           beta_env_b, beta_d1_W, beta_d1_b, beta_d2_W, beta_d2_b, emb_W0,
           emb_b0, emb_W1, emb_b1, emb_gamma_W, mlp_W0, mlp_b0, mlp_W1, mlp_b1,
           mlp_W2, mlp_b2, mp_proj_W, mp_proj_b, mp_gamma_W, orb_W, orb_b):
    E = N * K
    RW = 2 * N
    f32 = jnp.float32
    ind = ind_neighbour.astype(jnp.int32)
    idx = jnp.concatenate([ind[:, :8], ind[:, 8:]], axis=0)
    idxg = idx.T
    r8 = jnp.concatenate([r, jnp.zeros((N, 5), f32)], axis=1)
    r64 = jnp.tile(r8, (1, 8))
    w2 = jnp.concatenate(
        [weight_neighbour[:, :8], weight_neighbour[:, 8:]], axis=0)
    Rt8 = jnp.concatenate([R_orb.T, jnp.zeros((5, 32), f32)], axis=0)
    R2 = jnp.sum(R_orb * R_orb, axis=1)[None, :]

    tileb = lambda v, p: jnp.concatenate([v.reshape(1, -1)] * p, axis=1)
    row = lambda v: v.reshape(1, -1)
    cbc = _blockdiag(jnp.ones((8, 8), f32), 8)
    ob8 = _blockdiag(jnp.ones((1, 8), f32), 8)
    eyeb = _blockdiag(jnp.ones((8, 16), f32), 8)
    mask = jnp.zeros((1, 64), f32)
    for j in range(8):
        mask = mask.at[0, 8 * j + 3].set(1.0)
    d1W8 = jnp.concatenate([beta_d1_W, jnp.zeros((5, 16), f32)], axis=0)
    D1d = _blockdiag(d1W8, 8)
    D2d = _blockdiag(beta_d2_W, 8)
    EWd = _blockdiag(beta_env_W, 8)
    eW0_8 = jnp.concatenate([emb_W0, jnp.zeros((4, WIDTH), f32)], axis=0)
    W0d = _blockdiag(eW0_8, 8)
    EGd = _blockdiag(emb_gamma_W, 8)
    MGd = _blockdiag(mp_gamma_W, 8)

    rnbw = _sc_gather_wide(r8, idxg, 1000)

    slab0 = pl.BlockSpec((BNB, 64), lambda i: (i, 0))
    slab1 = pl.BlockSpec((BNB, 64), lambda i: (i + NBS, 0))
    slab0n = pl.BlockSpec((BNB, 8), lambda i: (i, 0))
    slab1n = pl.BlockSpec((BNB, 8), lambda i: (i + NBS, 0))
    beta0, beta1, hpre, h = pl.pallas_call(
        _t1_body,
        grid=(N // BNB,),
        in_specs=[
            pl.BlockSpec((BNB, 64), lambda i: (i, 0)),
            slab0, slab1, slab0n, slab1n,
            _full((8, 64)),
            _full((1, 16)), _full((64, 64)), _full((64, 128)), _full((1, 64)),
            _full((128, 64)), _full((1, 64)),
            _full((64, 128)), _full((1, 128)), _full((128, 64)), _full((1, 64)),
            _full((64, 512)), _full((1, 512)), _full((128, 128)),
            _full((1, 128)), _full((64, 512)),
            _full((WIDTH, 128)), _full((1, 128)),
            _full((WIDTH, WIDTH)), _full((1, WIDTH)),
            _full((WIDTH, WIDTH)), _full((1, WIDTH)),
        ],
        out_specs=[
            pl.BlockSpec((BNB, 64), lambda i: (i, 0)),
            pl.BlockSpec((BNB, 64), lambda i: (i, 0)),
            pl.BlockSpec((BNB, WIDTH), lambda i: (i, 0)),
            pl.BlockSpec((BNB, WIDTH), lambda i: (i, 0)),
        ],
        out_shape=[
            jax.ShapeDtypeStruct((N, 64), f32),
            jax.ShapeDtypeStruct((N, 64), f32),
            jax.ShapeDtypeStruct((N, WIDTH), f32),
            jax.ShapeDtypeStruct((N, WIDTH), f32),
        ],
        compiler_params=pltpu.CompilerParams(
            dimension_semantics=("arbitrary",)),
    )(r64, rnbw, rnbw, w2, w2, ob8,
      row(beta_scales), cbc, eyeb, mask,
      EWd, tileb(beta_env_b, 8), D1d, tileb(beta_d1_b, 8),
      D2d, tileb(beta_d2_b, 8),
      W0d, tileb(emb_b0, 8), _blockdiag(emb_W1, 2), tileb(emb_b1, 2), EGd,
      jnp.concatenate([mlp_W0, mp_proj_W], axis=1),
      jnp.concatenate([row(mlp_b0), row(mp_proj_b)], axis=1),
      mlp_W1, row(mlp_b1), mlp_W2, row(mlp_b2))

    hnbw = _sc_gather_wide(h, idxg, 200)

    slab0w = pl.BlockSpec((BNB, 512), lambda i: (i, 0))
    slab1w = pl.BlockSpec((BNB, 512), lambda i: (i + NBS, 0))
    nblk = pl.BlockSpec((BNB, 64), lambda i: (i, 0))
    phi = pl.pallas_call(
        _t2_body,
        grid=(N // BNB,),
        in_specs=[
            nblk, nblk, slab0w, slab1w,
            pl.BlockSpec((BNB, WIDTH), lambda i: (i, 0)),
            pl.BlockSpec((BNB, 8), lambda i: (i, 0)),
            _full((64, 512)), _full((8, 32)), _full((1, 32)),
            _full((WIDTH, 32)), _full((1, 32)),
        ],
        out_specs=pl.BlockSpec((BNB, 32), lambda i: (i, 0)),
        out_shape=jax.ShapeDtypeStruct((N, 32), f32),
        compiler_params=pltpu.CompilerParams(
            dimension_semantics=("arbitrary",)),
    )(beta0, beta1, hnbw, hnbw, hpre, r8,
      MGd, Rt8, R2, orb_W, row(orb_b))

    return phi


# speedup vs baseline: 1.2359x; 1.0208x over previous
import functools

import jax
import jax.numpy as jnp
from jax import lax
from jax.experimental import pallas as pl
from jax.experimental.pallas import tpu as pltpu
from jax.experimental.pallas import tpu_sc as plsc

N = 50000
K = 16
WIDTH = 64
CUTOFF = 3.0

BNA = 2000
BNB = 2000
CH1 = 5000
CH2 = 1000

NBS = N // BNB


def _sc_gather_wide(table, idxg, cw):
    M, D = table.shape
    R = idxg.shape[1]
    n_chunks = R // cw
    info = plsc.get_sparse_core_info()
    nw = info.num_cores * info.num_subcores
    n_iter = (n_chunks + nw - 1) // nw
    mesh = plsc.VectorSubcoreMesh(core_axis_name="c", subcore_axis_name="s")

    @functools.partial(
        pl.kernel,
        mesh=mesh,
        out_type=jax.ShapeDtypeStruct((R, 8 * D), jnp.float32),
        scratch_types=[
            pltpu.VMEM((8, cw), jnp.int32),
            pltpu.VMEM((8, cw, D), jnp.float32),
            pltpu.SemaphoreType.DMA,
        ],
        compiler_params=pltpu.CompilerParams(use_tc_tiling_on_sc=False),
    )
    def k(table_hbm, idxg_hbm, out_hbm, idx_v, rows_v, sem):
        wid = lax.axis_index("s") * info.num_cores + lax.axis_index("c")

        def body(i, carry):
            g = i * nw + wid

            @pl.when(g < n_chunks)
            def _():
                base = g * cw
                pltpu.sync_copy(idxg_hbm.at[:, pl.ds(base, cw)], idx_v)
                descs = [
                    pltpu.async_copy(table_hbm.at[idx_v.at[j]],
                                     rows_v.at[j], sem)
                    for j in range(8)
                ]
                for dsc in descs:
                    dsc.wait()
                for j in range(8):
                    pltpu.sync_copy(
                        rows_v.at[j],
                        out_hbm.at[pl.ds(base, cw), pl.ds(j * D, D)])

            return carry

        lax.fori_loop(0, n_iter, body, 0)

    return k(table, idxg)


def _silu(x):
    return x * (1.0 / (1.0 + jnp.exp(-x)))


def _t1_body(r64_ref, rnb0_ref, rnb1_ref, w20_ref, w21_ref, ob8_ref,
             scl_ref, cbc_ref, eyeb_ref, mask_ref, EWd_ref, envbd_ref,
             D1d_ref, d1bd_ref, D2d_ref, d2bd_ref,
             W0d_ref, b0d_ref, W1_ref, b1d_ref, EGd_ref,
             mW0_ref, mb0_ref, mW1_ref, mb1_ref, mW2_ref, mb2_ref,
             beta0_ref, beta1_ref, hpre_ref, h_ref):
    inv_s = 1.0 / jnp.log1p(jnp.exp(scl_ref[...]))
    inv_sP = jnp.concatenate([inv_s] * 8, axis=1)
    CCW = jnp.concatenate([cbc_ref[...], eyeb_ref[...] * inv_sP], axis=1)
    rcp = r64_ref[...]

    def slab(rnb_ref, w2_ref, beta_ref):
        diffp = rcp - rnb_ref[...]
        sq = diffp * diffp
        CC = sq @ CCW
        dist2bc = CC[:, :64]
        distbc = jnp.sqrt(dist2bc)
        env = jnp.exp(-CC[:, 64:])
        envp = env @ EWd_ref[...] + envbd_ref[...]

        d = _silu(diffp @ D1d_ref[...] + d1bd_ref[...])
        d = d @ D2d_ref[...] + d2bd_ref[...]

        q = distbc * (1.0 / CUTOFF)
        u = dist2bc * (1.0 / (CUTOFF * CUTOFF))
        u2 = u * u
        cut = jnp.where(u < 1.0,
                        1.0 + u2 * (24.0 * q - (15.0 + 10.0 * u)), 0.0)
        wv = w2_ref[...] @ ob8_ref[...]
        beta = envp * d * cut * wv
        beta_ref[...] = beta
        gp = diffp + distbc * mask_ref[...]

        x1 = _silu(gp @ W0d_ref[...] + b0d_ref[...])
        bg = beta @ EGd_ref[...]
        acc = None
        for j in range(4):
            x2 = _silu(x1[:, 128 * j:128 * (j + 1)] @ W1_ref[...]
                       + b1d_ref[...])
            p = x2 * bg[:, 128 * j:128 * (j + 1)]
            acc = p if acc is None else acc + p
        return acc

    s = slab(rnb0_ref, w20_ref, beta0_ref) + slab(rnb1_ref, w21_ref, beta1_ref)
    h0 = s[:, :WIDTH] + s[:, WIDTH:]

    y = h0 @ mW0_ref[...] + mb0_ref[...]
    h = _silu(y[:, :WIDTH])
    hpre_ref[...] = y[:, WIDTH:]
    h = _silu(h @ mW1_ref[...] + mb1_ref[...])
    h_ref[...] = h @ mW2_ref[...] + mb2_ref[...]


def _t2_body(b0_ref, b1_ref, hnb0_ref, hnb1_ref, hpre_ref, r_ref, MGd_ref,
             Rt_ref, R2_ref, orbW_ref, orbb_ref, phi_ref):
    def slab(beta_ref, hnb_ref):
        bg = beta_ref[...] @ MGd_ref[...]
        hv = hnb_ref[...]
        acc = None
        for j in range(4):
            p = bg[:, 128 * j:128 * (j + 1)] * hv[:, 128 * j:128 * (j + 1)]
            acc = p if acc is None else acc + p
        return acc

    s = slab(b0_ref, hnb0_ref) + slab(b1_ref, hnb1_ref)
    msum = s[:, :WIDTH] + s[:, WIDTH:]
    h_out = _silu(hpre_ref[...] + msum)

    rc = r_ref[...]
    rr = jnp.sum(rc * rc, axis=1, keepdims=True)
    d2 = rr - 2.0 * (rc @ Rt_ref[...]) + R2_ref[...]
    deo = jnp.sqrt(jnp.maximum(d2, 0.0))
    phi_ref[...] = (h_out @ orbW_ref[...] + orbb_ref[...]) * jnp.exp(-0.2 * deo)


def _full(shape):
    nd = len(shape)
    return pl.BlockSpec(shape, lambda *_: (0,) * nd)


def _blockdiag(W, p):
    a, b = W.shape
    out = jnp.zeros((p * a, p * b), jnp.float32)
    for j in range(p):
        out = out.at[j * a:(j + 1) * a, j * b:(j + 1) * b].set(W)
    return out


def kernel(r, ind_neighbour, weight_neighbour, R_orb, beta_scales, beta_env_W,
           beta_env_b, beta_d1_W, beta_d1_b, beta_d2_W, beta_d2_b, emb_W0,
           emb_b0, emb_W1, emb_b1, emb_gamma_W, mlp_W0, mlp_b0, mlp_W1, mlp_b1,
           mlp_W2, mlp_b2, mp_proj_W, mp_proj_b, mp_gamma_W, orb_W, orb_b):
    E = N * K
    RW = 2 * N
    f32 = jnp.float32
    ind = ind_neighbour.astype(jnp.int32)
    idx = jnp.concatenate([ind[:, :8], ind[:, 8:]], axis=0)
    idxg = idx.T
    r8 = jnp.concatenate([r, jnp.zeros((N, 5), f32)], axis=1)
    r64 = jnp.tile(r8, (1, 8))
    w2 = jnp.concatenate(
        [weight_neighbour[:, :8], weight_neighbour[:, 8:]], axis=0)
    Rt8 = jnp.concatenate([R_orb.T, jnp.zeros((5, 32), f32)], axis=0)
    R2 = jnp.sum(R_orb * R_orb, axis=1)[None, :]

    tileb = lambda v, p: jnp.concatenate([v.reshape(1, -1)] * p, axis=1)
    row = lambda v: v.reshape(1, -1)
    cbc = _blockdiag(jnp.ones((8, 8), f32), 8)
    ob8 = _blockdiag(jnp.ones((1, 8), f32), 8)
    eyeb = _blockdiag(jnp.ones((8, 16), f32), 8)
    mask = jnp.zeros((1, 64), f32)
    for j in range(8):
        mask = mask.at[0, 8 * j + 3].set(1.0)
    d1W8 = jnp.concatenate([beta_d1_W, jnp.zeros((5, 16), f32)], axis=0)
    D1d = _blockdiag(d1W8, 8)
    D2d = _blockdiag(beta_d2_W, 8)
    EWd = _blockdiag(beta_env_W, 8)
    eW0_8 = jnp.concatenate([emb_W0, jnp.zeros((4, WIDTH), f32)], axis=0)
    W0d = _blockdiag(eW0_8, 8)
    EGd = _blockdiag(emb_gamma_W, 8)
    MGd = _blockdiag(mp_gamma_W, 8)

    rnbw = _sc_gather_wide(r8, idxg, 1000)

    slab0 = pl.BlockSpec((BNB, 64), lambda i: (i, 0))
    slab1 = pl.BlockSpec((BNB, 64), lambda i: (i + NBS, 0))
    slab0n = pl.BlockSpec((BNB, 8), lambda i: (i, 0))
    slab1n = pl.BlockSpec((BNB, 8), lambda i: (i + NBS, 0))
    beta0, beta1, hpre, h = pl.pallas_call(
        _t1_body,
        grid=(N // BNB,),
        in_specs=[
            pl.BlockSpec((BNB, 64), lambda i: (i, 0)),
            slab0, slab1, slab0n, slab1n,
            _full((8, 64)),
            _full((1, 16)), _full((64, 64)), _full((64, 128)), _full((1, 64)),
            _full((128, 64)), _full((1, 64)),
            _full((64, 128)), _full((1, 128)), _full((128, 64)), _full((1, 64)),
            _full((64, 512)), _full((1, 512)), _full((128, 128)),
            _full((1, 128)), _full((64, 512)),
            _full((WIDTH, 128)), _full((1, 128)),
            _full((WIDTH, WIDTH)), _full((1, WIDTH)),
            _full((WIDTH, WIDTH)), _full((1, WIDTH)),
        ],
        out_specs=[
            pl.BlockSpec((BNB, 64), lambda i: (i, 0)),
            pl.BlockSpec((BNB, 64), lambda i: (i, 0)),
            pl.BlockSpec((BNB, WIDTH), lambda i: (i, 0)),
            pl.BlockSpec((BNB, WIDTH), lambda i: (i, 0)),
        ],
        out_shape=[
            jax.ShapeDtypeStruct((N, 64), f32),
            jax.ShapeDtypeStruct((N, 64), f32),
            jax.ShapeDtypeStruct((N, WIDTH), f32),
            jax.ShapeDtypeStruct((N, WIDTH), f32),
        ],
        compiler_params=pltpu.CompilerParams(
            dimension_semantics=("parallel",)),
    )(r64, rnbw, rnbw, w2, w2, ob8,
      row(beta_scales), cbc, eyeb, mask,
      EWd, tileb(beta_env_b, 8), D1d, tileb(beta_d1_b, 8),
      D2d, tileb(beta_d2_b, 8),
      W0d, tileb(emb_b0, 8), _blockdiag(emb_W1, 2), tileb(emb_b1, 2), EGd,
      jnp.concatenate([mlp_W0, mp_proj_W], axis=1),
      jnp.concatenate([row(mlp_b0), row(mp_proj_b)], axis=1),
      mlp_W1, row(mlp_b1), mlp_W2, row(mlp_b2))

    hnbw = _sc_gather_wide(h, idxg, 200)

    slab0w = pl.BlockSpec((BNB, 512), lambda i: (i, 0))
    slab1w = pl.BlockSpec((BNB, 512), lambda i: (i + NBS, 0))
    nblk = pl.BlockSpec((BNB, 64), lambda i: (i, 0))
    phi = pl.pallas_call(
        _t2_body,
        grid=(N // BNB,),
        in_specs=[
            nblk, nblk, slab0w, slab1w,
            pl.BlockSpec((BNB, WIDTH), lambda i: (i, 0)),
            pl.BlockSpec((BNB, 8), lambda i: (i, 0)),
            _full((64, 512)), _full((8, 32)), _full((1, 32)),
            _full((WIDTH, 32)), _full((1, 32)),
        ],
        out_specs=pl.BlockSpec((BNB, 32), lambda i: (i, 0)),
        out_shape=jax.ShapeDtypeStruct((N, 32), f32),
        compiler_params=pltpu.CompilerParams(
            dimension_semantics=("parallel",)),
    )(beta0, beta1, hnbw, hnbw, hpre, r8,
      MGd, Rt8, R2, orb_W, row(orb_b))

    return phi
